# baseline (speedup 1.0000x reference)
import jax
import jax.numpy as jnp
from jax.experimental import pallas as pl
from jax.experimental.pallas import tpu as pltpu


def _round_up(x, m):
    return ((x + m - 1) // m) * m


def _mlp_kernel(y_ref, w1_ref, b1_ref, w2_ref, b2_ref, w3_ref, b3_ref,
                w4_ref, o_ref):
    q = o_ref.shape[2]
    yb = y_ref[...].astype(jnp.bfloat16)
    yp = jnp.concatenate(
        [yb[0 * q:1 * q], yb[1 * q:2 * q], yb[2 * q:3 * q], yb[3 * q:4 * q]],
        axis=1)

    h = jnp.dot(yp, w1_ref[...], preferred_element_type=jnp.float32)
    h = jnp.maximum(h + b1_ref[...], 0.0).astype(jnp.bfloat16)

    h = jnp.dot(h, w2_ref[...], preferred_element_type=jnp.float32)
    h = jnp.maximum(h + b2_ref[...], 0.0).astype(jnp.bfloat16)

    h = jnp.dot(h, w3_ref[...], preferred_element_type=jnp.float32)
    h = jnp.maximum(h + b3_ref[...], 0.0).astype(jnp.bfloat16)

    o = jax.lax.dot_general(w4_ref[...], h, (((1,), (1,)), ((), ())),
                            preferred_element_type=jnp.float32)
    o_ref[...] = o[0:4, :][None]


def kernel(t, y, w1, b1, w2, b2, w3, b3, w4c, b4):
    del t
    N = y.shape[0]
    bn = 8192
    q = bn // 4

    n_pad = _round_up(N, bn)
    if n_pad != N:
        y = jnp.pad(y, ((0, n_pad - N), (0, 0)))
    nb = n_pad // bn

    eye4 = jnp.eye(4, dtype=jnp.float32)
    w1p = jnp.kron(eye4, w1.T).astype(jnp.bfloat16)
    w2p = jnp.kron(eye4, w2.T).astype(jnp.bfloat16)
    w3p = jnp.kron(eye4, w3.T).astype(jnp.bfloat16)
    w4p = jnp.pad(jnp.kron(eye4, w4c.T), ((0, 4), (0, 0)))
    w4p = w4p.astype(jnp.bfloat16)
    b1p = jnp.tile(b1.T, (1, 4))
    b2p = jnp.tile(b2.T, (1, 4))
    b3p = jnp.tile(b3.T, (1, 4))

    resident = lambda shape: pl.BlockSpec(shape, lambda i: tuple(0 for _ in shape))

    out3 = pl.pallas_call(
        _mlp_kernel,
        out_shape=jax.ShapeDtypeStruct((nb, 4, q), jnp.float32),
        grid=(nb,),
        in_specs=[
            pl.BlockSpec((bn, 5), lambda i: (i, 0)),
            resident((20, 256)), resident((1, 256)),
            resident((256, 256)), resident((1, 256)),
            resident((256, 256)), resident((1, 256)),
            resident((8, 256)),
        ],
        out_specs=pl.BlockSpec((1, 4, q), lambda i: (i, 0, 0)),
        compiler_params=pltpu.CompilerParams(
            dimension_semantics=("parallel",),
            vmem_limit_bytes=56 * 1024 * 1024),
    )(y, w1p, b1p, w2p, b2p, w3p, b3p, w4p)

    out = out3.reshape(n_pad, 1)[:N] + b4[0, 0]
    return out


# speedup vs baseline: 3.1696x; 3.1696x over previous
import jax
import jax.numpy as jnp
from jax.experimental import pallas as pl
from jax.experimental.pallas import tpu as pltpu


def _round_up(x, m):
    return ((x + m - 1) // m) * m


def _mlp_kernel(x_ref, w1_ref, w2_ref, w3_ref, w4_ref, b4_ref, o_ref):
    bn = x_ref.shape[1]
    x = x_ref[...]
    ones = jnp.ones((1, bn), dtype=jnp.bfloat16)

    h = jnp.dot(w1_ref[...], x, preferred_element_type=jnp.float32)
    h = jnp.maximum(h.astype(jnp.bfloat16), 0)

    h = jnp.concatenate([h, ones], axis=0)
    h = jnp.dot(w2_ref[...], h, preferred_element_type=jnp.float32)
    h = jnp.maximum(h.astype(jnp.bfloat16), 0)

    h = jnp.concatenate([h, ones], axis=0)
    h = jnp.dot(w3_ref[...], h, preferred_element_type=jnp.float32)
    h = jnp.maximum(h, 0.0)

    o_ref[...] = jnp.sum(h * w4_ref[...], axis=0, keepdims=True) + b4_ref[...]


def kernel(t, y, w1, b1, w2, b2, w3, b3, w4c, b4):
    del t
    N = y.shape[0]
    bn = 16384
    n_pad = _round_up(N, bn)
    nb = n_pad // bn

    y_t = jnp.concatenate([y.T, jnp.ones((1, N), jnp.float32)]).astype(jnp.bfloat16)
    y_t = jnp.pad(y_t, ((0, 0), (0, n_pad - N)))

    w1a = jnp.concatenate([w1, b1], axis=1).astype(jnp.bfloat16)
    w2a = jnp.concatenate([w2, b2], axis=1).astype(jnp.bfloat16)
    w3a = jnp.concatenate([w3, b3], axis=1).astype(jnp.bfloat16)

    resident = lambda shape: pl.BlockSpec(shape, lambda i: tuple(0 for _ in shape))

    out_t = pl.pallas_call(
        _mlp_kernel,
        out_shape=jax.ShapeDtypeStruct((1, n_pad), jnp.float32),
        grid=(nb,),
        in_specs=[
            pl.BlockSpec((6, bn), lambda i: (0, i)),
            resident((64, 6)),
            resident((64, 65)),
            resident((64, 65)),
            resident((64, 1)), resident((1, 1)),
        ],
        out_specs=pl.BlockSpec((1, bn), lambda i: (0, i)),
        compiler_params=pltpu.CompilerParams(
            dimension_semantics=("parallel",),
            vmem_limit_bytes=56 * 1024 * 1024),
    )(y_t, w1a, w2a, w3a, w4c, b4)

    return out_t[:, :N].T


# speedup vs baseline: 4.5042x; 1.4211x over previous
import jax
import jax.numpy as jnp
from jax.experimental import pallas as pl
from jax.experimental.pallas import tpu as pltpu


def _round_up(x, m):
    return ((x + m - 1) // m) * m


def _net_half(x, w1, w2, w3, w4, b4):
    bh = x.shape[1]
    ones = jnp.ones((1, bh), dtype=jnp.bfloat16)
    x = jnp.concatenate([x.astype(jnp.bfloat16), ones], axis=0)
    h = jnp.dot(w1, x, preferred_element_type=jnp.float32)
    h = jnp.maximum(h.astype(jnp.bfloat16), 0)
    h = jnp.concatenate([h, ones], axis=0)
    h = jnp.dot(w2, h, preferred_element_type=jnp.float32)
    h = jnp.maximum(h.astype(jnp.bfloat16), 0)
    h = jnp.concatenate([h, ones], axis=0)
    h = jnp.dot(w3, h, preferred_element_type=jnp.float32)
    h = jnp.maximum(h, 0.0)
    return jnp.sum(h * w4, axis=0, keepdims=True) + b4


def _mlp_kernel(x_ref, w1_ref, w2_ref, w3_ref, w4_ref, b4_ref, o_ref):
    bn = x_ref.shape[1]
    hb = bn // 64
    w1, w2, w3 = w1_ref[...], w2_ref[...], w3_ref[...]
    w4, b4 = w4_ref[...], b4_ref[...]
    outs = [_net_half(x_ref[:, i * hb:(i + 1) * hb], w1, w2, w3, w4, b4)
            for i in range(64)]
    o_ref[...] = jnp.concatenate(outs, axis=1)


def kernel(t, y, w1, b1, w2, b2, w3, b3, w4c, b4):
    del t
    N = y.shape[0]
    bn = 262144
    n_pad = _round_up(N, bn)
    nb = n_pad // bn

    y_t = jnp.pad(y.T, ((0, 0), (0, n_pad - N)))

    w1a = jnp.concatenate([w1, b1], axis=1).astype(jnp.bfloat16)
    w2a = jnp.concatenate([w2, b2], axis=1).astype(jnp.bfloat16)
    w3a = jnp.concatenate([w3, b3], axis=1).astype(jnp.bfloat16)

    resident = lambda shape: pl.BlockSpec(shape, lambda i: tuple(0 for _ in shape))

    out_t = pl.pallas_call(
        _mlp_kernel,
        out_shape=jax.ShapeDtypeStruct((1, n_pad), jnp.float32),
        grid=(nb,),
        in_specs=[
            pl.BlockSpec((5, bn), lambda i: (0, i)),
            resident((64, 6)),
            resident((64, 65)),
            resident((64, 65)),
            resident((64, 1)), resident((1, 1)),
        ],
        out_specs=pl.BlockSpec((1, bn), lambda i: (0, i)),
        compiler_params=pltpu.CompilerParams(
            dimension_semantics=("parallel",),
            vmem_limit_bytes=56 * 1024 * 1024),
    )(y_t, w1a, w2a, w3a, w4c, b4)

    return out_t[:, :N].T
